# Initial kernel scaffold; baseline (speedup 1.0000x reference)
import jax
import jax.numpy as jnp
from jax.experimental import pallas as pl


def kernel(input_ids, token_type_ids, features, token_type_table, ln_gamma, ln_beta):
    raise NotImplementedError("write your pallas kernel here")


# speedup vs baseline: 4.5403x; 4.5403x over previous
import jax
import jax.numpy as jnp
from jax.experimental import pallas as pl
from jax.experimental.pallas import tpu as pltpu

_EPS = 1e-12
_ROW_BLK = 2048


def _ln_body(ids_ref, feat_ref, table_ref, gamma_ref, beta_ref, out_ref):
    ids = ids_ref[0]
    ids_col = ids.reshape(_ROW_BLK, 1)
    feat = feat_ref[...]
    t0 = table_ref[0:1, :]
    t1 = table_ref[1:2, :]
    tte = jnp.where(ids_col == 1, t1, t0)
    emb = feat + tte
    mean = jnp.mean(emb, axis=-1, keepdims=True)
    centered = emb - mean
    var = jnp.mean(centered * centered, axis=-1, keepdims=True)
    inv = jax.lax.rsqrt(var + _EPS)
    out_ref[...] = (centered * inv) * gamma_ref[...] + beta_ref[...]


def kernel(input_ids, token_type_ids, features, token_type_table, ln_gamma, ln_beta):
    del input_ids
    B, S, D = features.shape
    rows = B * S
    nblk = rows // _ROW_BLK
    feat2 = features.reshape(rows, D)
    ids3 = token_type_ids.reshape(nblk, 1, _ROW_BLK).astype(jnp.int32)
    gamma2 = ln_gamma.reshape(1, D)
    beta2 = ln_beta.reshape(1, D)

    out = pl.pallas_call(
        _ln_body,
        grid=(nblk,),
        in_specs=[
            pl.BlockSpec((1, 1, _ROW_BLK), lambda i: (i, 0, 0)),
            pl.BlockSpec((_ROW_BLK, D), lambda i: (i, 0)),
            pl.BlockSpec((2, D), lambda i: (0, 0)),
            pl.BlockSpec((1, D), lambda i: (0, 0)),
            pl.BlockSpec((1, D), lambda i: (0, 0)),
        ],
        out_specs=pl.BlockSpec((_ROW_BLK, D), lambda i: (i, 0)),
        out_shape=jax.ShapeDtypeStruct((rows, D), jnp.float32),
        compiler_params=pltpu.CompilerParams(
            dimension_semantics=("arbitrary",),
        ),
    )(ids3, feat2, token_type_table, gamma2, beta2)
    return out.reshape(B, S, D)
